# baseline (speedup 1.0000x reference)
import functools

import jax
import jax.numpy as jnp
from jax import lax
from jax.experimental import pallas as pl
from jax.experimental.pallas import tpu as pltpu
from jax.experimental.pallas import tpu_sc as plsc

BATCH = 16384
K = 16
NC = 2
NS = 16
L = 16
NW = NC * NS
BPW = BATCH // NW
CHUNK = 128
NCHUNK = BPW // CHUNK

_mesh = plsc.VectorSubcoreMesh(core_axis_name="c", subcore_axis_name="s")


@functools.partial(
    pl.kernel,
    out_type=jax.ShapeDtypeStruct((BATCH,), jnp.float32),
    mesh=_mesh,
    compiler_params=pltpu.CompilerParams(
        needs_layout_passes=False, use_tc_tiling_on_sc=False
    ),
    scratch_types=[
        pltpu.VMEM((BPW, 2), jnp.int32),
        pltpu.VMEM((BPW,), jnp.int32),
        pltpu.VMEM((BPW,), jnp.int32),
        pltpu.VMEM((BPW, K), jnp.float32),
        pltpu.VMEM((BPW, K), jnp.float32),
        pltpu.VMEM((BPW,), jnp.float32),
        pltpu.SemaphoreType.DMA,
        pltpu.SemaphoreType.DMA,
    ],
)
def _mf_predict(x_hbm, w_hbm, h_hbm, out_hbm,
                xv, uidx, iidx, urows, vrows, outv, sem_u, sem_v):
    wid = lax.axis_index("s") * NC + lax.axis_index("c")
    base = wid * BPW
    iota = lax.iota(jnp.int32, L)
    zeros = jnp.zeros((L,), jnp.int32)
    ones = jnp.ones((L,), jnp.int32)

    pltpu.sync_copy(x_hbm.at[pl.ds(base, BPW)], xv)

    def split_body(g, carry):
        rowids = g * L + iota
        u = plsc.load_gather(xv, [rowids, zeros])
        v = plsc.load_gather(xv, [rowids, ones])
        plsc.store_scatter(uidx, [rowids], u)
        plsc.store_scatter(iidx, [rowids], v)
        return carry

    lax.fori_loop(0, BPW // L, split_body, 0)

    copies = []
    for c in range(NCHUNK):
        sl = pl.ds(c * CHUNK, CHUNK)
        copies.append(pltpu.async_copy(w_hbm.at[uidx.at[sl]], urows.at[sl], sem_u))
        copies.append(pltpu.async_copy(h_hbm.at[iidx.at[sl]], vrows.at[sl], sem_v))
    for cp in copies:
        cp.wait()

    def dot_body(g, carry):
        rowids = g * L + iota
        acc = jnp.zeros((L,), jnp.float32)
        for d in range(K):
            dsplat = jnp.full((L,), d, jnp.int32)
            u = plsc.load_gather(urows, [rowids, dsplat])
            v = plsc.load_gather(vrows, [rowids, dsplat])
            acc = acc + u * v
        sig = 1.0 / (1.0 + jnp.exp(-acc))
        plsc.store_scatter(outv, [rowids], sig)
        return carry

    lax.fori_loop(0, BPW // L, dot_body, 0)

    pltpu.sync_copy(outv, out_hbm.at[pl.ds(base, BPW)])


def kernel(x, W, H):
    return _mf_predict(x, W, H)
